# Initial kernel scaffold; baseline (speedup 1.0000x reference)
import jax
import jax.numpy as jnp
from jax.experimental import pallas as pl


def kernel(inputs, targets):
    raise NotImplementedError("write your pallas kernel here")


# speedup vs baseline: 3.2210x; 3.2210x over previous
import jax
import jax.numpy as jnp
from jax.experimental import pallas as pl

N = 2097152
EPS = 1e-07
RATIO = 0.3

_ROWS = 2048
_COLS = 1024
_BLK_ROWS = 256


def _loss_key_body(x_ref, z_ref, key_ref):
    x = x_ref[...]
    z = z_ref[...]
    probs = jnp.clip(jax.nn.sigmoid(x), EPS, 1.0 - EPS)
    pt = probs * z + (1.0 - probs) * (1.0 - z)
    bce = jnp.maximum(x, 0.0) - x * z + jnp.log1p(jnp.exp(-jnp.abs(x)))
    pos = z == 1.0
    one_m = 1.0 - pt
    focal = jnp.where(pos, one_m * one_m, one_m)
    alpha = jnp.where(pos, jnp.float32(1.0), jnp.float32(0.5))
    loss = alpha * focal * bce
    bits = jax.lax.bitcast_convert_type(loss, jnp.int32)
    key_ref[...] = jnp.where(pos, bits | jnp.int32(-(2**31)), bits)


def _compute_keys(inputs, targets):
    x2 = inputs.reshape(_ROWS, _COLS)
    z2 = targets.reshape(_ROWS, _COLS)
    grid = (_ROWS // _BLK_ROWS,)
    keys = pl.pallas_call(
        _loss_key_body,
        grid=grid,
        in_specs=[
            pl.BlockSpec((_BLK_ROWS, _COLS), lambda i: (i, 0)),
            pl.BlockSpec((_BLK_ROWS, _COLS), lambda i: (i, 0)),
        ],
        out_specs=pl.BlockSpec((_BLK_ROWS, _COLS), lambda i: (i, 0)),
        out_shape=jax.ShapeDtypeStruct((_ROWS, _COLS), jnp.int32),
    )(x2, z2)
    return keys.reshape(N)


def kernel(inputs, targets):
    keys = _compute_keys(inputs, targets)
    keys_u = jax.lax.bitcast_convert_type(keys, jnp.uint32)
    skey = jnp.sort(keys_u)
    n_pos = jnp.sum((keys_u >> 31).astype(jnp.int32))
    n_neg = jnp.int32(N) - n_pos
    k_pos = jnp.maximum(1, jnp.floor(n_pos.astype(jnp.float32) * RATIO).astype(jnp.int32))
    k_neg = jnp.maximum(1, jnp.floor(n_neg.astype(jnp.float32) * RATIO).astype(jnp.int32))
    alive_p = n_pos > 0
    alive_n = n_neg > 0
    k_pos = jnp.where(alive_p, k_pos, 0)
    k_neg = jnp.where(alive_n, k_neg, 0)
    loss_sorted = jax.lax.bitcast_convert_type(skey & jnp.uint32(0x7FFFFFFF), jnp.float32)
    idx = jnp.arange(N, dtype=jnp.int32)
    sel_pos = idx >= (N - k_pos)
    sel_neg = (idx >= (N - n_pos - k_neg)) & (idx < (N - n_pos))
    s = jnp.sum(jnp.where(sel_pos | sel_neg, loss_sorted, 0.0))
    return s / (k_pos + k_neg).astype(jnp.float32)


# speedup vs baseline: 13.2664x; 4.1188x over previous
import jax
import jax.numpy as jnp
from jax import lax
from jax.experimental import pallas as pl
from jax.experimental.pallas import tpu as pltpu
from jax.experimental.pallas import tpu_sc as plsc

N = 2097152
EPS = 1e-07
RATIO = 0.3

_ROWS = 2048
_COLS = 1024
_BLK_ROWS = 256

_NSUB = 16
_PER_SUB = N // _NSUB
_CHUNK = 8192
_NCHUNK = _PER_SUB // _CHUNK
_VECS = _CHUNK // 16
_HIST = 16 * 512

_TOPBIT = -(2**31)
_MASK31 = 0x7FFFFFFF


def _loss_key_body(x_ref, z_ref, key_ref):
    x = x_ref[...]
    z = z_ref[...]
    probs = jnp.clip(jax.nn.sigmoid(x), EPS, 1.0 - EPS)
    pt = probs * z + (1.0 - probs) * (1.0 - z)
    bce = jnp.maximum(x, 0.0) - x * z + jnp.log1p(jnp.exp(-jnp.abs(x)))
    pos = z == 1.0
    one_m = 1.0 - pt
    focal = jnp.where(pos, one_m * one_m, one_m)
    alpha = jnp.where(pos, jnp.float32(1.0), jnp.float32(0.5))
    loss = alpha * focal * bce
    bits = lax.bitcast_convert_type(loss, jnp.int32)
    key_ref[...] = jnp.where(pos, bits | _TOPBIT, bits)


def _compute_keys(inputs, targets):
    x2 = inputs.reshape(_ROWS, _COLS)
    z2 = targets.reshape(_ROWS, _COLS)
    keys = pl.pallas_call(
        _loss_key_body,
        grid=(_ROWS // _BLK_ROWS,),
        in_specs=[
            pl.BlockSpec((_BLK_ROWS, _COLS), lambda i: (i, 0)),
            pl.BlockSpec((_BLK_ROWS, _COLS), lambda i: (i, 0)),
        ],
        out_specs=pl.BlockSpec((_BLK_ROWS, _COLS), lambda i: (i, 0)),
        out_shape=jax.ShapeDtypeStruct((_ROWS, _COLS), jnp.int32),
    )(x2, z2)
    return keys.reshape(N)


def _sc_body(keys_hbm, out_hbm, buf, hist_c, hist_s, red_c, red_s,
             glob_c, glob_s, outv, stage_c, stage_s):
    sid = lax.axis_index("s")
    cid = lax.axis_index("c")
    wid = sid
    lane = lax.iota(jnp.int32, 16)
    lane_base = lane * jnp.int32(512)
    ones_i = jnp.ones((16,), jnp.int32)
    zeros_i = jnp.zeros((16,), jnp.int32)
    zeros_f = jnp.zeros((16,), jnp.float32)

    def _lane_i(v, j):
        return jnp.sum(jnp.where(lane == j, v, jnp.int32(0)))

    def _lane_f(v, j):
        return jnp.sum(jnp.where(lane == j, v, jnp.float32(0.0)))

    def _revcumsum(v):
        return lax.rev(jnp.cumsum(lax.rev(v, (0,))), (0,))

    P = [jnp.zeros((), jnp.int32), jnp.zeros((), jnp.int32) + _TOPBIT]
    c_above = [jnp.zeros((), jnp.int32), jnp.zeros((), jnp.int32)]
    s_above = [jnp.zeros((), jnp.float32), jnp.zeros((), jnp.float32)]
    kk = [jnp.zeros((), jnp.int32), jnp.zeros((), jnp.int32)]
    alive = [None, None]

    for l in range(4):
        shift = 24 - 8 * l
        mask_hi = _TOPBIT if l == 0 else jnp.int32(-(1 << (32 - 8 * l)))
        shift_v = jnp.full((16,), shift, jnp.int32)

        def _zero(i, _):
            hist_c[pl.ds(i * 16, 16)] = zeros_i
            hist_s[pl.ds(i * 16, 16)] = zeros_f
            return 0

        lax.fori_loop(0, _HIST // 16, _zero, 0)

        Pn, Pp = P[0], P[1]

        def _chunk(c, _):
            base = wid * _PER_SUB + c * _CHUNK
            pltpu.sync_copy(keys_hbm.at[pl.ds(base, _CHUNK)], buf)

            def _scan(i, _):
                x = buf[pl.ds(i * 16, 16)]
                mn = ((x ^ Pn) & mask_hi) == 0
                mp = ((x ^ Pp) & mask_hi) == 0
                b = lax.shift_right_logical(x, shift_v) & jnp.int32(0xFF)
                idx = lane_base + b + jnp.where(mp, jnp.int32(256), jnp.int32(0))
                m = mn | mp
                plsc.addupdate_scatter(hist_c, [idx], ones_i, mask=m)
                loss = plsc.bitcast(x & _MASK31, jnp.float32)
                plsc.addupdate_scatter(hist_s, [idx], loss, mask=m)
                return 0

            lax.fori_loop(0, _VECS, _scan, 0)
            return 0

        lax.fori_loop(0, _NCHUNK, _chunk, 0)

        def _lred(j, _):
            def _acc(ln, carry):
                ac, asum = carry
                off = ln * jnp.int32(512) + j * 16
                return ac + hist_c[pl.ds(off, 16)], asum + hist_s[pl.ds(off, 16)]

            ac, asum = lax.fori_loop(0, 16, _acc, (zeros_i, zeros_f))
            red_c[pl.ds(j * 16, 16)] = ac
            red_s[pl.ds(j * 16, 16)] = asum
            return 0

        lax.fori_loop(0, 32, _lred, 0)

        pltpu.sync_copy(red_c, stage_c.at[wid])
        pltpu.sync_copy(red_s, stage_s.at[wid])
        plsc.subcore_barrier()

        pltpu.sync_copy(stage_c, glob_c)
        pltpu.sync_copy(stage_s, glob_s)
        plsc.subcore_barrier()

        def _gagg(j, _):
            def _acc(s, carry):
                ac, asum = carry
                return ac + glob_c[s, pl.ds(j * 16, 16)], asum + glob_s[s, pl.ds(j * 16, 16)]

            ac, asum = lax.fori_loop(0, 16, _acc, (zeros_i, zeros_f))
            red_c[pl.ds(j * 16, 16)] = ac
            red_s[pl.ds(j * 16, 16)] = asum
            return 0

        lax.fori_loop(0, 32, _gagg, 0)

        for cls in range(2):
            base = cls * 256
            chunkV = zeros_i
            chunkS = zeros_f
            for ci in range(16):
                vc = red_c[pl.ds(base + ci * 16, 16)]
                vs = red_s[pl.ds(base + ci * 16, 16)]
                chunkV = jnp.where(lane == ci, jnp.sum(vc), chunkV)
                chunkS = jnp.where(lane == ci, jnp.sum(vs), chunkS)
            if l == 0:
                n_cls = jnp.sum(chunkV)
                alive[cls] = n_cls > 0
                kf = n_cls.astype(jnp.float32) * jnp.float32(RATIO)
                kk[cls] = jnp.maximum(jnp.int32(1), kf.astype(jnp.int32))
            r = kk[cls] - c_above[cls]
            SCi = _revcumsum(chunkV)
            SSi = _revcumsum(chunkS)
            I = jnp.maximum(jnp.max(plsc.all_reduce_population_count(SCi >= r)) - 1, 0)
            A_c = _lane_i(SCi - chunkV, I)
            A_s = _lane_f(SSi - chunkS, I)
            c16 = red_c[pl.ds(base + I * 16, 16)]
            s16 = red_s[pl.ds(base + I * 16, 16)]
            W = _revcumsum(c16)
            Ws = _revcumsum(s16)
            jj = jnp.maximum(jnp.max(plsc.all_reduce_population_count((A_c + W) >= r)) - 1, 0)
            B = I * 16 + jj
            cn = c_above[cls] + A_c + _lane_i(W - c16, jj)
            sn = s_above[cls] + A_s + _lane_f(Ws - s16, jj)
            pn = P[cls] | lax.shift_left(B, jnp.int32(shift))
            c_above[cls] = jnp.where(alive[cls], cn, c_above[cls])
            s_above[cls] = jnp.where(alive[cls], sn, s_above[cls])
            P[cls] = jnp.where(alive[cls], pn, P[cls])

    num = zeros_f
    den = jnp.float32(0.0)
    for cls in range(2):
        t_bits = zeros_i + (P[cls] & _MASK31)
        t_f = plsc.bitcast(t_bits, jnp.float32)
        contrib = s_above[cls] + (kk[cls] - c_above[cls]).astype(jnp.float32) * t_f
        af = alive[cls].astype(jnp.float32)
        num = num + af * contrib
        den = den + af * kk[cls].astype(jnp.float32)
    outv[...] = num / den

    @pl.when((sid == 0) & (cid == 0))
    def _():
        pltpu.sync_copy(outv, out_hbm)


def _sc_select(keys):
    mesh = plsc.VectorSubcoreMesh(core_axis_name="c", subcore_axis_name="s",
                                  num_cores=1)
    f = pl.kernel(
        _sc_body,
        out_type=jax.ShapeDtypeStruct((16,), jnp.float32),
        mesh=mesh,
        compiler_params=pltpu.CompilerParams(needs_layout_passes=False),
        scratch_types=[
            pltpu.VMEM((_CHUNK,), jnp.int32),
            pltpu.VMEM((_HIST,), jnp.int32),
            pltpu.VMEM((_HIST,), jnp.float32),
            pltpu.VMEM((512,), jnp.int32),
            pltpu.VMEM((512,), jnp.float32),
            pltpu.VMEM((16, 512), jnp.int32),
            pltpu.VMEM((16, 512), jnp.float32),
            pltpu.VMEM((16,), jnp.float32),
            pltpu.VMEM_SHARED((16, 512), jnp.int32),
            pltpu.VMEM_SHARED((16, 512), jnp.float32),
        ],
    )
    return f(keys)


def kernel(inputs, targets):
    keys = _compute_keys(inputs, targets)
    out = _sc_select(keys)
    return out[0]


# speedup vs baseline: 20.7032x; 1.5606x over previous
import jax
import jax.numpy as jnp
from jax import lax
from jax.experimental import pallas as pl
from jax.experimental.pallas import tpu as pltpu
from jax.experimental.pallas import tpu_sc as plsc

N = 2097152
EPS = 1e-07
RATIO = 0.3

_ROWS = 2048
_COLS = 1024
_BLK_ROWS = 256

_NSUB = 16
_PER_SUB = N // _NSUB
_CHUNK = 8192
_NCHUNK = _PER_SUB // _CHUNK
_VECS = _CHUNK // 16
_HIST = 16 * 512

_TOPBIT = -(2**31)
_MASK31 = 0x7FFFFFFF


def _loss_key_body(x_ref, z_ref, key_ref):
    x = x_ref[...]
    z = z_ref[...]
    probs = jnp.clip(jax.nn.sigmoid(x), EPS, 1.0 - EPS)
    pt = probs * z + (1.0 - probs) * (1.0 - z)
    bce = jnp.maximum(x, 0.0) - x * z + jnp.log1p(jnp.exp(-jnp.abs(x)))
    pos = z == 1.0
    one_m = 1.0 - pt
    focal = jnp.where(pos, one_m * one_m, one_m)
    alpha = jnp.where(pos, jnp.float32(1.0), jnp.float32(0.5))
    loss = alpha * focal * bce
    bits = lax.bitcast_convert_type(loss, jnp.int32)
    key_ref[...] = jnp.where(pos, bits | _TOPBIT, bits)


def _compute_keys(inputs, targets):
    x2 = inputs.reshape(_ROWS, _COLS)
    z2 = targets.reshape(_ROWS, _COLS)
    keys = pl.pallas_call(
        _loss_key_body,
        grid=(_ROWS // _BLK_ROWS,),
        in_specs=[
            pl.BlockSpec((_BLK_ROWS, _COLS), lambda i: (i, 0)),
            pl.BlockSpec((_BLK_ROWS, _COLS), lambda i: (i, 0)),
        ],
        out_specs=pl.BlockSpec((_BLK_ROWS, _COLS), lambda i: (i, 0)),
        out_shape=jax.ShapeDtypeStruct((_ROWS, _COLS), jnp.int32),
    )(x2, z2)
    return keys.reshape(N)


_UNROLL = 4


def _sc_body(keys_hbm, out_hbm, buf0, buf1, hist_c, hist_s, red_c, red_s,
             glob_c, glob_s, outv, stage_c, stage_s, sem0, sem1):
    sid = lax.axis_index("s")
    cid = lax.axis_index("c")
    wid = sid
    lane = lax.iota(jnp.int32, 16)
    lane_base = lane * jnp.int32(512)
    ones_i = jnp.ones((16,), jnp.int32)
    zeros_i = jnp.zeros((16,), jnp.int32)
    zeros_f = jnp.zeros((16,), jnp.float32)
    ubase = [lane_base + u * 8192 for u in range(_UNROLL)]

    def _lane_i(v, j):
        return jnp.sum(jnp.where(lane == j, v, jnp.int32(0)))

    def _lane_f(v, j):
        return jnp.sum(jnp.where(lane == j, v, jnp.float32(0.0)))

    def _revcumsum(v):
        return lax.rev(jnp.cumsum(lax.rev(v, (0,))), (0,))

    P = [jnp.zeros((), jnp.int32), jnp.zeros((), jnp.int32) + _TOPBIT]
    c_above = [jnp.zeros((), jnp.int32), jnp.zeros((), jnp.int32)]
    s_above = [jnp.zeros((), jnp.float32), jnp.zeros((), jnp.float32)]
    kk = [jnp.zeros((), jnp.int32), jnp.zeros((), jnp.int32)]
    alive = [None, None]

    for l in range(4):
        shift = 24 - 8 * l
        mask_hi = _TOPBIT if l == 0 else jnp.int32(-(1 << (32 - 8 * l)))
        shift_v = jnp.full((16,), shift, jnp.int32)

        def _zero(i, _):
            hist_c[pl.ds(i * 16, 16)] = zeros_i
            hist_s[pl.ds(i * 16, 16)] = zeros_f
            return 0

        lax.fori_loop(0, _UNROLL * _HIST // 16, _zero, 0)

        Pn, Pp = P[0], P[1]

        def _scan_buf(buf):
            def _scan(i, _):
                vo = i * (16 * _UNROLL)
                idxs, losses, ms = [], [], []
                for u in range(_UNROLL):
                    x = buf[pl.ds(vo + u * 16, 16)]
                    mn = ((x ^ Pn) & mask_hi) == 0
                    mp = ((x ^ Pp) & mask_hi) == 0
                    b = lax.shift_right_logical(x, shift_v) & jnp.int32(0xFF)
                    idxs.append(ubase[u] + b
                                + jnp.where(mp, jnp.int32(256), jnp.int32(0)))
                    ms.append(mn | mp)
                    losses.append(plsc.bitcast(x & _MASK31, jnp.float32))
                for u in range(_UNROLL):
                    plsc.addupdate_scatter(hist_c, [idxs[u]], ones_i, mask=ms[u])
                    plsc.addupdate_scatter(hist_s, [idxs[u]], losses[u], mask=ms[u])
                return 0

            lax.fori_loop(0, _VECS // _UNROLL, _scan, 0)

        def _chunk_slice(c):
            return keys_hbm.at[pl.ds(wid * _PER_SUB + c * _CHUNK, _CHUNK)]

        pltpu.async_copy(_chunk_slice(0), buf0, sem0)

        def _dbl(j, _):
            pltpu.async_copy(_chunk_slice(2 * j + 1), buf1, sem1)
            pltpu.make_async_copy(_chunk_slice(0), buf0, sem0).wait()
            _scan_buf(buf0)
            pltpu.async_copy(_chunk_slice(jnp.minimum(2 * j + 2, _NCHUNK - 1)),
                             buf0, sem0)
            pltpu.make_async_copy(_chunk_slice(0), buf1, sem1).wait()
            _scan_buf(buf1)
            return 0

        lax.fori_loop(0, _NCHUNK // 2, _dbl, 0)
        pltpu.make_async_copy(_chunk_slice(0), buf0, sem0).wait()

        def _lred(j, _):
            def _acc(m, carry):
                ac, asum = carry
                off = (m // 16) * jnp.int32(_HIST) + (m % 16) * jnp.int32(512) + j * 16
                return ac + hist_c[pl.ds(off, 16)], asum + hist_s[pl.ds(off, 16)]

            ac, asum = lax.fori_loop(0, 16 * _UNROLL, _acc, (zeros_i, zeros_f))
            red_c[pl.ds(j * 16, 16)] = ac
            red_s[pl.ds(j * 16, 16)] = asum
            return 0

        lax.fori_loop(0, 32, _lred, 0)

        pltpu.sync_copy(red_c, stage_c.at[wid])
        pltpu.sync_copy(red_s, stage_s.at[wid])
        plsc.subcore_barrier()

        pltpu.sync_copy(stage_c, glob_c)
        pltpu.sync_copy(stage_s, glob_s)
        plsc.subcore_barrier()

        def _gagg(j, _):
            def _acc(s, carry):
                ac, asum = carry
                return ac + glob_c[s, pl.ds(j * 16, 16)], asum + glob_s[s, pl.ds(j * 16, 16)]

            ac, asum = lax.fori_loop(0, 16, _acc, (zeros_i, zeros_f))
            red_c[pl.ds(j * 16, 16)] = ac
            red_s[pl.ds(j * 16, 16)] = asum
            return 0

        lax.fori_loop(0, 32, _gagg, 0)

        for cls in range(2):
            base = cls * 256
            chunkV = zeros_i
            chunkS = zeros_f
            for ci in range(16):
                vc = red_c[pl.ds(base + ci * 16, 16)]
                vs = red_s[pl.ds(base + ci * 16, 16)]
                chunkV = jnp.where(lane == ci, jnp.sum(vc), chunkV)
                chunkS = jnp.where(lane == ci, jnp.sum(vs), chunkS)
            if l == 0:
                n_cls = jnp.sum(chunkV)
                alive[cls] = n_cls > 0
                kf = n_cls.astype(jnp.float32) * jnp.float32(RATIO)
                kk[cls] = jnp.maximum(jnp.int32(1), kf.astype(jnp.int32))
            r = kk[cls] - c_above[cls]
            SCi = _revcumsum(chunkV)
            SSi = _revcumsum(chunkS)
            I = jnp.maximum(jnp.max(plsc.all_reduce_population_count(SCi >= r)) - 1, 0)
            A_c = _lane_i(SCi - chunkV, I)
            A_s = _lane_f(SSi - chunkS, I)
            c16 = red_c[pl.ds(base + I * 16, 16)]
            s16 = red_s[pl.ds(base + I * 16, 16)]
            W = _revcumsum(c16)
            Ws = _revcumsum(s16)
            jj = jnp.maximum(jnp.max(plsc.all_reduce_population_count((A_c + W) >= r)) - 1, 0)
            B = I * 16 + jj
            cn = c_above[cls] + A_c + _lane_i(W - c16, jj)
            sn = s_above[cls] + A_s + _lane_f(Ws - s16, jj)
            pn = P[cls] | lax.shift_left(B, jnp.int32(shift))
            c_above[cls] = jnp.where(alive[cls], cn, c_above[cls])
            s_above[cls] = jnp.where(alive[cls], sn, s_above[cls])
            P[cls] = jnp.where(alive[cls], pn, P[cls])

    num = zeros_f
    den = jnp.float32(0.0)
    for cls in range(2):
        t_bits = zeros_i + (P[cls] & _MASK31)
        t_f = plsc.bitcast(t_bits, jnp.float32)
        contrib = s_above[cls] + (kk[cls] - c_above[cls]).astype(jnp.float32) * t_f
        af = alive[cls].astype(jnp.float32)
        num = num + af * contrib
        den = den + af * kk[cls].astype(jnp.float32)
    outv[...] = num / den

    @pl.when((sid == 0) & (cid == 0))
    def _():
        pltpu.sync_copy(outv, out_hbm)


def _sc_select(keys):
    mesh = plsc.VectorSubcoreMesh(core_axis_name="c", subcore_axis_name="s",
                                  num_cores=1)
    f = pl.kernel(
        _sc_body,
        out_type=jax.ShapeDtypeStruct((16,), jnp.float32),
        mesh=mesh,
        compiler_params=pltpu.CompilerParams(needs_layout_passes=False),
        scratch_types=[
            pltpu.VMEM((_CHUNK,), jnp.int32),
            pltpu.VMEM((_CHUNK,), jnp.int32),
            pltpu.VMEM((_UNROLL * _HIST,), jnp.int32),
            pltpu.VMEM((_UNROLL * _HIST,), jnp.float32),
            pltpu.VMEM((512,), jnp.int32),
            pltpu.VMEM((512,), jnp.float32),
            pltpu.VMEM((16, 512), jnp.int32),
            pltpu.VMEM((16, 512), jnp.float32),
            pltpu.VMEM((16,), jnp.float32),
            pltpu.VMEM_SHARED((16, 512), jnp.int32),
            pltpu.VMEM_SHARED((16, 512), jnp.float32),
            pltpu.SemaphoreType.DMA,
            pltpu.SemaphoreType.DMA,
        ],
    )
    return f(keys)


def kernel(inputs, targets):
    keys = _compute_keys(inputs, targets)
    out = _sc_select(keys)
    return out[0]


# speedup vs baseline: 26.3420x; 1.2724x over previous
import functools

import jax
import jax.numpy as jnp
from jax import lax
from jax.experimental import pallas as pl
from jax.experimental.pallas import tpu as pltpu
from jax.experimental.pallas import tpu_sc as plsc

N = 2097152
EPS = 1e-07
RATIO = 0.3

_ROWS = 2048
_COLS = 1024
_BLK_ROWS = 256

_NCORE = 2
_NSUB = 16
_NWORK = _NCORE * _NSUB
_PER_SUB = N // _NWORK
_CHUNK = 8192
_NCHUNK = _PER_SUB // _CHUNK
_VECS = _CHUNK // 16
_HIST = 16 * 512
_UNROLL = 4

_TOPBIT = -(2**31)
_MASK31 = 0x7FFFFFFF


def _loss_key_body(x_ref, z_ref, key_ref):
    x = x_ref[...]
    z = z_ref[...]
    probs = jnp.clip(jax.nn.sigmoid(x), EPS, 1.0 - EPS)
    pt = probs * z + (1.0 - probs) * (1.0 - z)
    bce = jnp.maximum(x, 0.0) - x * z + jnp.log1p(jnp.exp(-jnp.abs(x)))
    pos = z == 1.0
    one_m = 1.0 - pt
    focal = jnp.where(pos, one_m * one_m, one_m)
    alpha = jnp.where(pos, jnp.float32(1.0), jnp.float32(0.5))
    loss = alpha * focal * bce
    bits = lax.bitcast_convert_type(loss, jnp.int32)
    key_ref[...] = jnp.where(pos, bits | _TOPBIT, bits)


def _compute_keys(inputs, targets):
    x2 = inputs.reshape(_ROWS, _COLS)
    z2 = targets.reshape(_ROWS, _COLS)
    keys = pl.pallas_call(
        _loss_key_body,
        grid=(_ROWS // _BLK_ROWS,),
        in_specs=[
            pl.BlockSpec((_BLK_ROWS, _COLS), lambda i: (i, 0)),
            pl.BlockSpec((_BLK_ROWS, _COLS), lambda i: (i, 0)),
        ],
        out_specs=pl.BlockSpec((_BLK_ROWS, _COLS), lambda i: (i, 0)),
        out_shape=jax.ShapeDtypeStruct((_ROWS, _COLS), jnp.int32),
    )(x2, z2)
    return keys.reshape(N)


def _lane_i(lane, v, j):
    return jnp.sum(jnp.where(lane == j, v, jnp.int32(0)))


def _lane_f(lane, v, j):
    return jnp.sum(jnp.where(lane == j, v, jnp.float32(0.0)))


def _revcumsum(v):
    return lax.rev(jnp.cumsum(lax.rev(v, (0,))), (0,))


def _select_level(l, lane, red_c, red_s, st):
    P, c_above, s_above, kk, alive = [list(x) for x in st]
    shift = 24 - 8 * l
    zeros_i = jnp.zeros((16,), jnp.int32)
    zeros_f = jnp.zeros((16,), jnp.float32)
    for cls in range(2):
        base = cls * 256
        chunkV = zeros_i
        chunkS = zeros_f
        for ci in range(16):
            vc = red_c[pl.ds(base + ci * 16, 16)]
            vs = red_s[pl.ds(base + ci * 16, 16)]
            chunkV = jnp.where(lane == ci, jnp.sum(vc), chunkV)
            chunkS = jnp.where(lane == ci, jnp.sum(vs), chunkS)
        if l == 0:
            n_cls = jnp.sum(chunkV)
            alive[cls] = (n_cls > 0).astype(jnp.int32)
            kf = n_cls.astype(jnp.float32) * jnp.float32(RATIO)
            kk[cls] = jnp.maximum(jnp.int32(1), kf.astype(jnp.int32))
        r = kk[cls] - c_above[cls]
        SCi = _revcumsum(chunkV)
        SSi = _revcumsum(chunkS)
        I = jnp.maximum(jnp.max(plsc.all_reduce_population_count(SCi >= r)) - 1, 0)
        A_c = _lane_i(lane, SCi - chunkV, I)
        A_s = _lane_f(lane, SSi - chunkS, I)
        c16 = red_c[pl.ds(base + I * 16, 16)]
        s16 = red_s[pl.ds(base + I * 16, 16)]
        W = _revcumsum(c16)
        Ws = _revcumsum(s16)
        jj = jnp.maximum(jnp.max(plsc.all_reduce_population_count((A_c + W) >= r)) - 1, 0)
        B = I * 16 + jj
        cn = c_above[cls] + A_c + _lane_i(lane, W - c16, jj)
        sn = s_above[cls] + A_s + _lane_f(lane, Ws - s16, jj)
        pn = P[cls] | lax.shift_left(B, jnp.int32(shift))
        ok = alive[cls] > 0
        c_above[cls] = jnp.where(ok, cn, c_above[cls])
        s_above[cls] = jnp.where(ok, sn, s_above[cls])
        P[cls] = jnp.where(ok, pn, P[cls])
    return P, c_above, s_above, kk, alive


def _merge_prev(prev_c, prev_s, mc, ms, red_c, red_s):
    pltpu.sync_copy(prev_c, mc)
    pltpu.sync_copy(prev_s, ms)

    def _m(j, _):
        red_c[pl.ds(j * 16, 16)] = mc[0, pl.ds(j * 16, 16)] + mc[1, pl.ds(j * 16, 16)]
        red_s[pl.ds(j * 16, 16)] = ms[0, pl.ds(j * 16, 16)] + ms[1, pl.ds(j * 16, 16)]
        return 0

    lax.fori_loop(0, 32, _m, 0)


def _unpack_state(lane, stv_i, stv_f):
    vi = stv_i[...]
    vf = stv_f[...]
    P = [_lane_i(lane, vi, 0), _lane_i(lane, vi, 1)]
    c_above = [_lane_i(lane, vi, 2), _lane_i(lane, vi, 3)]
    kk = [_lane_i(lane, vi, 4), _lane_i(lane, vi, 5)]
    alive = [_lane_i(lane, vi, 6), _lane_i(lane, vi, 7)]
    s_above = [_lane_f(lane, vf, 0), _lane_f(lane, vf, 1)]
    return P, c_above, s_above, kk, alive


def _pack_state(lane, st):
    P, c_above, s_above, kk, alive = st
    vals_i = [P[0], P[1], c_above[0], c_above[1], kk[0], kk[1], alive[0], alive[1]]
    vi = jnp.zeros((16,), jnp.int32)
    for j, v in enumerate(vals_i):
        vi = jnp.where(lane == j, v, vi)
    vf = jnp.zeros((16,), jnp.float32)
    vf = jnp.where(lane == 0, s_above[0], vf)
    vf = jnp.where(lane == 1, s_above[1], vf)
    return vi, vf


def _init_state():
    z = jnp.zeros((), jnp.int32)
    zf = jnp.zeros((), jnp.float32)
    return ([z, z + _TOPBIT], [z, z], [zf, zf], [z, z], [z, z])


def _scan_body(l, keys_hbm, prev_c, prev_s, st_i_in, st_f_in,
               out_c, out_s, st_i_out, st_f_out,
               buf0, buf1, hist_c, hist_s, red_c, red_s, mp_c, mp_s, mc, ms,
               stv_i, stv_f, stage_c, stage_s, sem0, sem1):
    sid = lax.axis_index("s")
    cid = lax.axis_index("c")
    wid = cid * _NSUB + sid
    lane = lax.iota(jnp.int32, 16)
    lane_base = lane * jnp.int32(512)
    ones_i = jnp.ones((16,), jnp.int32)
    zeros_i = jnp.zeros((16,), jnp.int32)
    zeros_f = jnp.zeros((16,), jnp.float32)
    ubase = [lane_base + u * _HIST for u in range(_UNROLL)]

    if l == 0:
        st = _init_state()
    else:
        _merge_prev(prev_c, prev_s, mp_c, mp_s, red_c, red_s)
        if l == 1:
            st = _init_state()
        else:
            pltpu.sync_copy(st_i_in.at[0], stv_i)
            pltpu.sync_copy(st_f_in.at[0], stv_f)
            st = _unpack_state(lane, stv_i, stv_f)
        st = _select_level(l - 1, lane, red_c, red_s, st)
        vi, vf = _pack_state(lane, st)
        stv_i[...] = vi
        stv_f[...] = vf

    P = st[0]
    shift = 24 - 8 * l
    mask_hi = _TOPBIT if l == 0 else -(1 << (32 - 8 * l))
    shift_v = jnp.full((16,), shift, jnp.int32)

    def _zero(i, _):
        hist_c[pl.ds(i * 16, 16)] = zeros_i
        hist_s[pl.ds(i * 16, 16)] = zeros_f
        return 0

    lax.fori_loop(0, _UNROLL * _HIST // 16, _zero, 0)

    Pn, Pp = P[0], P[1]

    def _scan_buf(buf):
        def _scan(i, _):
            vo = i * (16 * _UNROLL)
            idxs, losses, ms_ = [], [], []
            for u in range(_UNROLL):
                x = buf[pl.ds(vo + u * 16, 16)]
                mn = ((x ^ Pn) & mask_hi) == 0
                mp = ((x ^ Pp) & mask_hi) == 0
                b = lax.shift_right_logical(x, shift_v) & jnp.int32(0xFF)
                idxs.append(ubase[u] + b
                            + jnp.where(mp, jnp.int32(256), jnp.int32(0)))
                ms_.append(mn | mp)
                losses.append(plsc.bitcast(x & _MASK31, jnp.float32))
            for u in range(_UNROLL):
                plsc.addupdate_scatter(hist_c, [idxs[u]], ones_i, mask=ms_[u])
                plsc.addupdate_scatter(hist_s, [idxs[u]], losses[u], mask=ms_[u])
            return 0

        lax.fori_loop(0, _VECS // _UNROLL, _scan, 0)

    def _chunk_slice(c):
        return keys_hbm.at[pl.ds(wid * _PER_SUB + c * _CHUNK, _CHUNK)]

    pltpu.async_copy(_chunk_slice(0), buf0, sem0)

    def _dbl(j, _):
        pltpu.async_copy(_chunk_slice(2 * j + 1), buf1, sem1)
        pltpu.make_async_copy(_chunk_slice(0), buf0, sem0).wait()
        _scan_buf(buf0)
        pltpu.async_copy(_chunk_slice(jnp.minimum(2 * j + 2, _NCHUNK - 1)),
                         buf0, sem0)
        pltpu.make_async_copy(_chunk_slice(0), buf1, sem1).wait()
        _scan_buf(buf1)
        return 0

    lax.fori_loop(0, _NCHUNK // 2, _dbl, 0)
    pltpu.make_async_copy(_chunk_slice(0), buf0, sem0).wait()

    def _fold(i, _):
        o = i * 16
        hist_c[pl.ds(o, 16)] = (hist_c[pl.ds(o, 16)] + hist_c[pl.ds(o + _HIST, 16)]
                                + hist_c[pl.ds(o + 2 * _HIST, 16)]
                                + hist_c[pl.ds(o + 3 * _HIST, 16)])
        hist_s[pl.ds(o, 16)] = (hist_s[pl.ds(o, 16)] + hist_s[pl.ds(o + _HIST, 16)]
                                + hist_s[pl.ds(o + 2 * _HIST, 16)]
                                + hist_s[pl.ds(o + 3 * _HIST, 16)])
        return 0

    lax.fori_loop(0, _HIST // 16, _fold, 0)

    def _lred(j, _):
        def _acc(ln, carry):
            ac, asum = carry
            off = ln * jnp.int32(512) + j * 16
            return ac + hist_c[pl.ds(off, 16)], asum + hist_s[pl.ds(off, 16)]

        ac, asum = lax.fori_loop(0, 16, _acc, (zeros_i, zeros_f))
        red_c[pl.ds(j * 16, 16)] = ac
        red_s[pl.ds(j * 16, 16)] = asum
        return 0

    lax.fori_loop(0, 32, _lred, 0)

    pltpu.sync_copy(red_c, stage_c.at[sid])
    pltpu.sync_copy(red_s, stage_s.at[sid])
    plsc.subcore_barrier()

    @pl.when(sid == 0)
    def _():
        def _gagg(j, _):
            def _acc(s, carry):
                ac, asum = carry
                return (ac + mc[s, pl.ds(j * 16, 16)],
                        asum + ms[s, pl.ds(j * 16, 16)])

            ac, asum = lax.fori_loop(0, 16, _acc, (zeros_i, zeros_f))
            red_c[pl.ds(j * 16, 16)] = ac
            red_s[pl.ds(j * 16, 16)] = asum
            return 0

        pltpu.sync_copy(stage_c, mc)
        pltpu.sync_copy(stage_s, ms)
        lax.fori_loop(0, 32, _gagg, 0)
        pltpu.sync_copy(red_c, out_c.at[cid])
        pltpu.sync_copy(red_s, out_s.at[cid])
        if l > 0:
            pltpu.sync_copy(stv_i, st_i_out.at[cid])
            pltpu.sync_copy(stv_f, st_f_out.at[cid])


def _final_body(prev_c, prev_s, st_i_in, st_f_in, out_hbm,
                red_c, red_s, mc, ms, stv_i, stv_f, outv):
    sid = lax.axis_index("s")
    cid = lax.axis_index("c")
    lane = lax.iota(jnp.int32, 16)

    @pl.when((sid == 0) & (cid == 0))
    def _():
        _merge_prev(prev_c, prev_s, mc, ms, red_c, red_s)
        pltpu.sync_copy(st_i_in.at[0], stv_i)
        pltpu.sync_copy(st_f_in.at[0], stv_f)
        st = _unpack_state(lane, stv_i, stv_f)
        P, c_above, s_above, kk, alive = _select_level(3, lane, red_c, red_s, st)
        num = jnp.zeros((16,), jnp.float32)
        den = jnp.zeros((), jnp.float32)
        for cls in range(2):
            t_bits = jnp.zeros((16,), jnp.int32) + (P[cls] & _MASK31)
            t_f = plsc.bitcast(t_bits, jnp.float32)
            contrib = s_above[cls] + (kk[cls] - c_above[cls]).astype(jnp.float32) * t_f
            af = alive[cls].astype(jnp.float32)
            num = num + af * contrib
            den = den + af * kk[cls].astype(jnp.float32)
        outv[...] = num / den
        pltpu.sync_copy(outv, out_hbm)


def _sc_select(keys):
    mesh = plsc.VectorSubcoreMesh(core_axis_name="c", subcore_axis_name="s",
                                  num_cores=_NCORE)
    params = pltpu.CompilerParams(needs_layout_passes=False)
    hist_out = (jax.ShapeDtypeStruct((_NCORE, 512), jnp.int32),
                jax.ShapeDtypeStruct((_NCORE, 512), jnp.float32))
    st_out = (jax.ShapeDtypeStruct((_NCORE, 16), jnp.int32),
              jax.ShapeDtypeStruct((_NCORE, 16), jnp.float32))
    scan_scratch = [
        pltpu.VMEM((_CHUNK,), jnp.int32),
        pltpu.VMEM((_CHUNK,), jnp.int32),
        pltpu.VMEM((_UNROLL * _HIST,), jnp.int32),
        pltpu.VMEM((_UNROLL * _HIST,), jnp.float32),
        pltpu.VMEM((512,), jnp.int32),
        pltpu.VMEM((512,), jnp.float32),
        pltpu.VMEM((2, 512), jnp.int32),
        pltpu.VMEM((2, 512), jnp.float32),
        pltpu.VMEM((16, 512), jnp.int32),
        pltpu.VMEM((16, 512), jnp.float32),
        pltpu.VMEM((16,), jnp.int32),
        pltpu.VMEM((16,), jnp.float32),
        pltpu.VMEM_SHARED((16, 512), jnp.int32),
        pltpu.VMEM_SHARED((16, 512), jnp.float32),
        pltpu.SemaphoreType.DMA,
        pltpu.SemaphoreType.DMA,
    ]

    zc = jnp.zeros((_NCORE, 512), jnp.int32)
    zs = jnp.zeros((_NCORE, 512), jnp.float32)
    zi = jnp.zeros((_NCORE, 16), jnp.int32)
    zf = jnp.zeros((_NCORE, 16), jnp.float32)

    hc, hs = None, None
    sti, stf = zi, zf
    for l in range(4):
        f = pl.kernel(
            functools.partial(_scan_body, l),
            out_type=hist_out + st_out,
            mesh=mesh,
            compiler_params=params,
            scratch_types=scan_scratch,
        )
        hc, hs, sti_n, stf_n = f(keys,
                                 zc if hc is None else hc,
                                 zs if hs is None else hs,
                                 sti, stf)
        if l > 0:
            sti, stf = sti_n, stf_n

    f = pl.kernel(
        _final_body,
        out_type=jax.ShapeDtypeStruct((16,), jnp.float32),
        mesh=mesh,
        compiler_params=params,
        scratch_types=[
            pltpu.VMEM((512,), jnp.int32),
            pltpu.VMEM((512,), jnp.float32),
            pltpu.VMEM((2, 512), jnp.int32),
            pltpu.VMEM((2, 512), jnp.float32),
            pltpu.VMEM((16,), jnp.int32),
            pltpu.VMEM((16,), jnp.float32),
            pltpu.VMEM((16,), jnp.float32),
        ],
    )
    return f(hc, hs, sti, stf)


def kernel(inputs, targets):
    keys = _compute_keys(inputs, targets)
    out = _sc_select(keys)
    return out[0]


# speedup vs baseline: 28.5714x; 1.0846x over previous
import functools

import jax
import jax.numpy as jnp
from jax import lax
from jax.experimental import pallas as pl
from jax.experimental.pallas import tpu as pltpu
from jax.experimental.pallas import tpu_sc as plsc

N = 2097152
EPS = 1e-07
RATIO = 0.3

_ROWS = 2048
_COLS = 1024
_BLK_ROWS = 256

_NCORE = 2
_NSUB = 16
_NWORK = _NCORE * _NSUB
_PER_SUB = N // _NWORK
_CHUNK = 8192
_NCHUNK = _PER_SUB // _CHUNK
_VECS = _CHUNK // 16
_HIST = 16 * 512
_UNROLL = 4

_TOPBIT = -(2**31)
_MASK31 = 0x7FFFFFFF


def _loss_key_body(x_ref, z_ref, key_ref):
    x = x_ref[...]
    z = z_ref[...]
    probs = jnp.clip(jax.nn.sigmoid(x), EPS, 1.0 - EPS)
    pt = probs * z + (1.0 - probs) * (1.0 - z)
    bce = jnp.maximum(x, 0.0) - x * z + jnp.log1p(jnp.exp(-jnp.abs(x)))
    pos = z == 1.0
    one_m = 1.0 - pt
    focal = jnp.where(pos, one_m * one_m, one_m)
    alpha = jnp.where(pos, jnp.float32(1.0), jnp.float32(0.5))
    loss = alpha * focal * bce
    bits = lax.bitcast_convert_type(loss, jnp.int32)
    key_ref[...] = jnp.where(pos, bits | _TOPBIT, bits)


def _compute_keys(inputs, targets):
    blk = N // 8
    keys = pl.pallas_call(
        _loss_key_body,
        grid=(8,),
        in_specs=[
            pl.BlockSpec((blk,), lambda i: (i,)),
            pl.BlockSpec((blk,), lambda i: (i,)),
        ],
        out_specs=pl.BlockSpec((blk,), lambda i: (i,)),
        out_shape=jax.ShapeDtypeStruct((N,), jnp.int32),
    )(inputs, targets)
    return keys


def _lane_i(lane, v, j):
    return jnp.sum(jnp.where(lane == j, v, jnp.int32(0)))


def _lane_f(lane, v, j):
    return jnp.sum(jnp.where(lane == j, v, jnp.float32(0.0)))


def _revcumsum(v):
    return lax.rev(jnp.cumsum(lax.rev(v, (0,))), (0,))


def _select_level(l, lane, red_c, red_s, st):
    P, c_above, s_above, kk, alive = [list(x) for x in st]
    shift = 24 - 8 * l
    zeros_i = jnp.zeros((16,), jnp.int32)
    zeros_f = jnp.zeros((16,), jnp.float32)
    for cls in range(2):
        base = cls * 256
        chunkV = zeros_i
        chunkS = zeros_f
        for ci in range(16):
            vc = red_c[pl.ds(base + ci * 16, 16)]
            vs = red_s[pl.ds(base + ci * 16, 16)]
            chunkV = jnp.where(lane == ci, jnp.sum(vc), chunkV)
            chunkS = jnp.where(lane == ci, jnp.sum(vs), chunkS)
        if l == 0:
            n_cls = jnp.sum(chunkV)
            alive[cls] = (n_cls > 0).astype(jnp.int32)
            kf = n_cls.astype(jnp.float32) * jnp.float32(RATIO)
            kk[cls] = jnp.maximum(jnp.int32(1), kf.astype(jnp.int32))
        r = kk[cls] - c_above[cls]
        SCi = _revcumsum(chunkV)
        SSi = _revcumsum(chunkS)
        I = jnp.maximum(jnp.max(plsc.all_reduce_population_count(SCi >= r)) - 1, 0)
        A_c = _lane_i(lane, SCi - chunkV, I)
        A_s = _lane_f(lane, SSi - chunkS, I)
        c16 = red_c[pl.ds(base + I * 16, 16)]
        s16 = red_s[pl.ds(base + I * 16, 16)]
        W = _revcumsum(c16)
        Ws = _revcumsum(s16)
        jj = jnp.maximum(jnp.max(plsc.all_reduce_population_count((A_c + W) >= r)) - 1, 0)
        B = I * 16 + jj
        cn = c_above[cls] + A_c + _lane_i(lane, W - c16, jj)
        sn = s_above[cls] + A_s + _lane_f(lane, Ws - s16, jj)
        pn = P[cls] | lax.shift_left(B, jnp.int32(shift))
        ok = alive[cls] > 0
        c_above[cls] = jnp.where(ok, cn, c_above[cls])
        s_above[cls] = jnp.where(ok, sn, s_above[cls])
        P[cls] = jnp.where(ok, pn, P[cls])
    return P, c_above, s_above, kk, alive


def _merge_prev(prev_c, prev_s, mc, ms, red_c, red_s):
    pltpu.sync_copy(prev_c, mc)
    pltpu.sync_copy(prev_s, ms)

    def _m(j, _):
        red_c[pl.ds(j * 16, 16)] = mc[0, pl.ds(j * 16, 16)] + mc[1, pl.ds(j * 16, 16)]
        red_s[pl.ds(j * 16, 16)] = ms[0, pl.ds(j * 16, 16)] + ms[1, pl.ds(j * 16, 16)]
        return 0

    lax.fori_loop(0, 32, _m, 0)


def _unpack_state(lane, stv_i, stv_f):
    vi = stv_i[...]
    vf = stv_f[...]
    P = [_lane_i(lane, vi, 0), _lane_i(lane, vi, 1)]
    c_above = [_lane_i(lane, vi, 2), _lane_i(lane, vi, 3)]
    kk = [_lane_i(lane, vi, 4), _lane_i(lane, vi, 5)]
    alive = [_lane_i(lane, vi, 6), _lane_i(lane, vi, 7)]
    s_above = [_lane_f(lane, vf, 0), _lane_f(lane, vf, 1)]
    return P, c_above, s_above, kk, alive


def _pack_state(lane, st):
    P, c_above, s_above, kk, alive = st
    vals_i = [P[0], P[1], c_above[0], c_above[1], kk[0], kk[1], alive[0], alive[1]]
    vi = jnp.zeros((16,), jnp.int32)
    for j, v in enumerate(vals_i):
        vi = jnp.where(lane == j, v, vi)
    vf = jnp.zeros((16,), jnp.float32)
    vf = jnp.where(lane == 0, s_above[0], vf)
    vf = jnp.where(lane == 1, s_above[1], vf)
    return vi, vf


def _init_state():
    z = jnp.zeros((), jnp.int32)
    zf = jnp.zeros((), jnp.float32)
    return ([z, z + _TOPBIT], [z, z], [zf, zf], [z, z], [z, z])


def _scan_body(l, keys_hbm, prev_c, prev_s, st_i_in, st_f_in,
               out_c, out_s, st_i_out, st_f_out,
               buf0, buf1, hist_c, hist_s, red_c, red_s, mp_c, mp_s, mc, ms,
               stv_i, stv_f, stage_c, stage_s, sem0, sem1):
    sid = lax.axis_index("s")
    cid = lax.axis_index("c")
    wid = cid * _NSUB + sid
    lane = lax.iota(jnp.int32, 16)
    lane_base = lane * jnp.int32(512)
    ones_i = jnp.ones((16,), jnp.int32)
    zeros_i = jnp.zeros((16,), jnp.int32)
    zeros_f = jnp.zeros((16,), jnp.float32)
    ubase = [lane_base + u * _HIST for u in range(_UNROLL)]

    if l == 0:
        st = _init_state()
    else:
        _merge_prev(prev_c, prev_s, mp_c, mp_s, red_c, red_s)
        if l == 1:
            st = _init_state()
        else:
            pltpu.sync_copy(st_i_in.at[0], stv_i)
            pltpu.sync_copy(st_f_in.at[0], stv_f)
            st = _unpack_state(lane, stv_i, stv_f)
        st = _select_level(l - 1, lane, red_c, red_s, st)
        vi, vf = _pack_state(lane, st)
        stv_i[...] = vi
        stv_f[...] = vf

    P = st[0]
    shift = 24 - 8 * l
    mask_hi = _TOPBIT if l == 0 else -(1 << (32 - 8 * l))
    shift_v = jnp.full((16,), shift, jnp.int32)

    def _zero(i, _):
        hist_c[pl.ds(i * 16, 16)] = zeros_i
        hist_s[pl.ds(i * 16, 16)] = zeros_f
        return 0

    lax.fori_loop(0, _UNROLL * _HIST // 16, _zero, 0)

    Pn, Pp = P[0], P[1]

    def _scan_buf(buf):
        def _scan(i, _):
            vo = i * (16 * _UNROLL)
            idxs, losses, ms_ = [], [], []
            for u in range(_UNROLL):
                x = buf[pl.ds(vo + u * 16, 16)]
                mn = ((x ^ Pn) & mask_hi) == 0
                mp = ((x ^ Pp) & mask_hi) == 0
                b = lax.shift_right_logical(x, shift_v) & jnp.int32(0xFF)
                idxs.append(ubase[u] + b
                            + jnp.where(mp, jnp.int32(256), jnp.int32(0)))
                ms_.append(mn | mp)
                losses.append(plsc.bitcast(x & _MASK31, jnp.float32))
            for u in range(_UNROLL):
                plsc.addupdate_scatter(hist_c, [idxs[u]], ones_i, mask=ms_[u])
                plsc.addupdate_scatter(hist_s, [idxs[u]], losses[u], mask=ms_[u])
            return 0

        lax.fori_loop(0, _VECS // _UNROLL, _scan, 0)

    def _chunk_slice(c):
        return keys_hbm.at[pl.ds(wid * _PER_SUB + c * _CHUNK, _CHUNK)]

    pltpu.async_copy(_chunk_slice(0), buf0, sem0)

    def _dbl(j, _):
        pltpu.async_copy(_chunk_slice(2 * j + 1), buf1, sem1)
        pltpu.make_async_copy(_chunk_slice(0), buf0, sem0).wait()
        _scan_buf(buf0)
        pltpu.async_copy(_chunk_slice(jnp.minimum(2 * j + 2, _NCHUNK - 1)),
                         buf0, sem0)
        pltpu.make_async_copy(_chunk_slice(0), buf1, sem1).wait()
        _scan_buf(buf1)
        return 0

    lax.fori_loop(0, _NCHUNK // 2, _dbl, 0)
    pltpu.make_async_copy(_chunk_slice(0), buf0, sem0).wait()

    def _fold(i, _):
        o = i * 16
        hist_c[pl.ds(o, 16)] = (hist_c[pl.ds(o, 16)] + hist_c[pl.ds(o + _HIST, 16)]
                                + hist_c[pl.ds(o + 2 * _HIST, 16)]
                                + hist_c[pl.ds(o + 3 * _HIST, 16)])
        hist_s[pl.ds(o, 16)] = (hist_s[pl.ds(o, 16)] + hist_s[pl.ds(o + _HIST, 16)]
                                + hist_s[pl.ds(o + 2 * _HIST, 16)]
                                + hist_s[pl.ds(o + 3 * _HIST, 16)])
        return 0

    lax.fori_loop(0, _HIST // 16, _fold, 0)

    def _lred(j, _):
        def _acc(ln, carry):
            ac, asum = carry
            off = ln * jnp.int32(512) + j * 16
            return ac + hist_c[pl.ds(off, 16)], asum + hist_s[pl.ds(off, 16)]

        ac, asum = lax.fori_loop(0, 16, _acc, (zeros_i, zeros_f))
        red_c[pl.ds(j * 16, 16)] = ac
        red_s[pl.ds(j * 16, 16)] = asum
        return 0

    lax.fori_loop(0, 32, _lred, 0)

    pltpu.sync_copy(red_c, stage_c.at[sid])
    pltpu.sync_copy(red_s, stage_s.at[sid])
    plsc.subcore_barrier()

    @pl.when(sid == 0)
    def _():
        def _gagg(j, _):
            def _acc(s, carry):
                ac, asum = carry
                return (ac + mc[s, pl.ds(j * 16, 16)],
                        asum + ms[s, pl.ds(j * 16, 16)])

            ac, asum = lax.fori_loop(0, 16, _acc, (zeros_i, zeros_f))
            red_c[pl.ds(j * 16, 16)] = ac
            red_s[pl.ds(j * 16, 16)] = asum
            return 0

        pltpu.sync_copy(stage_c, mc)
        pltpu.sync_copy(stage_s, ms)
        lax.fori_loop(0, 32, _gagg, 0)
        pltpu.sync_copy(red_c, out_c.at[cid])
        pltpu.sync_copy(red_s, out_s.at[cid])
        if l > 0:
            pltpu.sync_copy(stv_i, st_i_out.at[cid])
            pltpu.sync_copy(stv_f, st_f_out.at[cid])


def _final_body(prev_c, prev_s, st_i_in, st_f_in, out_hbm,
                red_c, red_s, mc, ms, stv_i, stv_f, outv):
    sid = lax.axis_index("s")
    cid = lax.axis_index("c")
    lane = lax.iota(jnp.int32, 16)

    @pl.when((sid == 0) & (cid == 0))
    def _():
        _merge_prev(prev_c, prev_s, mc, ms, red_c, red_s)
        pltpu.sync_copy(st_i_in.at[0], stv_i)
        pltpu.sync_copy(st_f_in.at[0], stv_f)
        st = _unpack_state(lane, stv_i, stv_f)
        P, c_above, s_above, kk, alive = _select_level(3, lane, red_c, red_s, st)
        num = jnp.zeros((16,), jnp.float32)
        den = jnp.zeros((), jnp.float32)
        for cls in range(2):
            t_bits = jnp.zeros((16,), jnp.int32) + (P[cls] & _MASK31)
            t_f = plsc.bitcast(t_bits, jnp.float32)
            contrib = s_above[cls] + (kk[cls] - c_above[cls]).astype(jnp.float32) * t_f
            af = alive[cls].astype(jnp.float32)
            num = num + af * contrib
            den = den + af * kk[cls].astype(jnp.float32)
        outv[...] = num / den
        pltpu.sync_copy(outv, out_hbm)


def _sc_select(keys):
    mesh = plsc.VectorSubcoreMesh(core_axis_name="c", subcore_axis_name="s",
                                  num_cores=_NCORE)
    params = pltpu.CompilerParams(needs_layout_passes=False)
    hist_out = (jax.ShapeDtypeStruct((_NCORE, 512), jnp.int32),
                jax.ShapeDtypeStruct((_NCORE, 512), jnp.float32))
    st_out = (jax.ShapeDtypeStruct((_NCORE, 16), jnp.int32),
              jax.ShapeDtypeStruct((_NCORE, 16), jnp.float32))
    scan_scratch = [
        pltpu.VMEM((_CHUNK,), jnp.int32),
        pltpu.VMEM((_CHUNK,), jnp.int32),
        pltpu.VMEM((_UNROLL * _HIST,), jnp.int32),
        pltpu.VMEM((_UNROLL * _HIST,), jnp.float32),
        pltpu.VMEM((512,), jnp.int32),
        pltpu.VMEM((512,), jnp.float32),
        pltpu.VMEM((2, 512), jnp.int32),
        pltpu.VMEM((2, 512), jnp.float32),
        pltpu.VMEM((16, 512), jnp.int32),
        pltpu.VMEM((16, 512), jnp.float32),
        pltpu.VMEM((16,), jnp.int32),
        pltpu.VMEM((16,), jnp.float32),
        pltpu.VMEM_SHARED((16, 512), jnp.int32),
        pltpu.VMEM_SHARED((16, 512), jnp.float32),
        pltpu.SemaphoreType.DMA,
        pltpu.SemaphoreType.DMA,
    ]

    zc = jnp.zeros((_NCORE, 512), jnp.int32)
    zs = jnp.zeros((_NCORE, 512), jnp.float32)
    zi = jnp.zeros((_NCORE, 16), jnp.int32)
    zf = jnp.zeros((_NCORE, 16), jnp.float32)

    hc, hs = None, None
    sti, stf = zi, zf
    for l in range(4):
        f = pl.kernel(
            functools.partial(_scan_body, l),
            out_type=hist_out + st_out,
            mesh=mesh,
            compiler_params=params,
            scratch_types=scan_scratch,
        )
        hc, hs, sti_n, stf_n = f(keys,
                                 zc if hc is None else hc,
                                 zs if hs is None else hs,
                                 sti, stf)
        if l > 0:
            sti, stf = sti_n, stf_n

    f = pl.kernel(
        _final_body,
        out_type=jax.ShapeDtypeStruct((16,), jnp.float32),
        mesh=mesh,
        compiler_params=params,
        scratch_types=[
            pltpu.VMEM((512,), jnp.int32),
            pltpu.VMEM((512,), jnp.float32),
            pltpu.VMEM((2, 512), jnp.int32),
            pltpu.VMEM((2, 512), jnp.float32),
            pltpu.VMEM((16,), jnp.int32),
            pltpu.VMEM((16,), jnp.float32),
            pltpu.VMEM((16,), jnp.float32),
        ],
    )
    return f(hc, hs, sti, stf)


def kernel(inputs, targets):
    keys = _compute_keys(inputs, targets)
    out = _sc_select(keys)
    return out[0]


# speedup vs baseline: 30.4153x; 1.0645x over previous
import functools

import jax
import jax.numpy as jnp
from jax import lax
from jax.experimental import pallas as pl
from jax.experimental.pallas import tpu as pltpu
from jax.experimental.pallas import tpu_sc as plsc

N = 2097152
EPS = 1e-07
RATIO = 0.3

_NCORE = 2
_NSUB = 16
_NWORK = _NCORE * _NSUB
_PER_SUB = N // _NWORK
_CHUNK = 8192
_NCHUNK = _PER_SUB // _CHUNK
_VECS = _CHUNK // 16
_HIST = 16 * 512
_UNROLL = 4

_TOPBIT = -(2**31)
_MASK31 = 0x7FFFFFFF


def _loss_key_body(x_ref, z_ref, key_ref):
    x = x_ref[...]
    z = z_ref[...]
    probs = jnp.clip(jax.nn.sigmoid(x), EPS, 1.0 - EPS)
    pt = probs * z + (1.0 - probs) * (1.0 - z)
    bce = jnp.maximum(x, 0.0) - x * z + jnp.log1p(jnp.exp(-jnp.abs(x)))
    pos = z == 1.0
    one_m = 1.0 - pt
    focal = jnp.where(pos, one_m * one_m, one_m)
    alpha = jnp.where(pos, jnp.float32(1.0), jnp.float32(0.5))
    loss = alpha * focal * bce
    bits = lax.bitcast_convert_type(loss, jnp.int32)
    key_ref[...] = jnp.where(pos, bits | _TOPBIT, bits)


def _compute_keys(inputs, targets):
    blk = N // 8
    return pl.pallas_call(
        _loss_key_body,
        grid=(8,),
        in_specs=[
            pl.BlockSpec((blk,), lambda i: (i,)),
            pl.BlockSpec((blk,), lambda i: (i,)),
        ],
        out_specs=pl.BlockSpec((blk,), lambda i: (i,)),
        out_shape=jax.ShapeDtypeStruct((N,), jnp.int32),
    )(inputs, targets)


def _lane_i(lane, v, j):
    return jnp.sum(jnp.where(lane == j, v, jnp.int32(0)))


def _lane_f(lane, v, j):
    return jnp.sum(jnp.where(lane == j, v, jnp.float32(0.0)))


def _revcumsum(v):
    return lax.rev(jnp.cumsum(lax.rev(v, (0,))), (0,))


def _select_level(l, lane, red_c, st):
    P, c_above, kk, alive = [list(x) for x in st]
    shift = 24 - 8 * l
    zeros_i = jnp.zeros((16,), jnp.int32)
    for cls in range(2):
        base = cls * 256
        chunkV = zeros_i
        for ci in range(16):
            vc = red_c[pl.ds(base + ci * 16, 16)]
            chunkV = jnp.where(lane == ci, jnp.sum(vc), chunkV)
        if l == 0:
            n_cls = jnp.sum(chunkV)
            alive[cls] = (n_cls > 0).astype(jnp.int32)
            kf = n_cls.astype(jnp.float32) * jnp.float32(RATIO)
            kk[cls] = jnp.maximum(jnp.int32(1), kf.astype(jnp.int32))
        r = kk[cls] - c_above[cls]
        SCi = _revcumsum(chunkV)
        I = jnp.maximum(jnp.max(plsc.all_reduce_population_count(SCi >= r)) - 1, 0)
        A_c = _lane_i(lane, SCi - chunkV, I)
        c16 = red_c[pl.ds(base + I * 16, 16)]
        W = _revcumsum(c16)
        jj = jnp.maximum(jnp.max(plsc.all_reduce_population_count((A_c + W) >= r)) - 1, 0)
        B = I * 16 + jj
        cn = c_above[cls] + A_c + _lane_i(lane, W - c16, jj)
        pn = P[cls] | lax.shift_left(B, jnp.int32(shift))
        ok = alive[cls] > 0
        c_above[cls] = jnp.where(ok, cn, c_above[cls])
        P[cls] = jnp.where(ok, pn, P[cls])
    return P, c_above, kk, alive


def _merge_prev(prev_c, mp_c, red_c):
    pltpu.sync_copy(prev_c, mp_c)

    def _m(j, _):
        red_c[pl.ds(j * 16, 16)] = (mp_c[0, pl.ds(j * 16, 16)]
                                    + mp_c[1, pl.ds(j * 16, 16)])
        return 0

    lax.fori_loop(0, 32, _m, 0)


def _unpack_state(lane, stv_i):
    vi = stv_i[...]
    P = [_lane_i(lane, vi, 0), _lane_i(lane, vi, 1)]
    c_above = [_lane_i(lane, vi, 2), _lane_i(lane, vi, 3)]
    kk = [_lane_i(lane, vi, 4), _lane_i(lane, vi, 5)]
    alive = [_lane_i(lane, vi, 6), _lane_i(lane, vi, 7)]
    return P, c_above, kk, alive


def _pack_state(lane, st):
    P, c_above, kk, alive = st
    vals_i = [P[0], P[1], c_above[0], c_above[1], kk[0], kk[1], alive[0], alive[1]]
    vi = jnp.zeros((16,), jnp.int32)
    for j, v in enumerate(vals_i):
        vi = jnp.where(lane == j, v, vi)
    return vi


def _init_state():
    z = jnp.zeros((), jnp.int32)
    return ([z, z + _TOPBIT], [z, z], [z, z], [z, z])


def _stream_chunks(keys_hbm, wid, buf0, buf1, sem0, sem1, scan_one, carry0):
    def _chunk_slice(c):
        return keys_hbm.at[pl.ds(wid * _PER_SUB + c * _CHUNK, _CHUNK)]

    pltpu.async_copy(_chunk_slice(0), buf0, sem0)

    def _dbl(j, carry):
        pltpu.async_copy(_chunk_slice(2 * j + 1), buf1, sem1)
        pltpu.make_async_copy(_chunk_slice(0), buf0, sem0).wait()
        carry = scan_one(buf0, carry)
        pltpu.async_copy(_chunk_slice(jnp.minimum(2 * j + 2, _NCHUNK - 1)),
                         buf0, sem0)
        pltpu.make_async_copy(_chunk_slice(0), buf1, sem1).wait()
        carry = scan_one(buf1, carry)
        return carry

    carry = lax.fori_loop(0, _NCHUNK // 2, _dbl, carry0)
    pltpu.make_async_copy(_chunk_slice(0), buf0, sem0).wait()
    return carry


def _scan_body(l, keys_hbm, prev_c, st_i_in, out_c, st_i_out,
               buf0, buf1, hist_c, red_c, mp_c, mc,
               stv_i, stage_c, sem0, sem1):
    sid = lax.axis_index("s")
    cid = lax.axis_index("c")
    wid = cid * _NSUB + sid
    lane = lax.iota(jnp.int32, 16)
    lane_base = lane * jnp.int32(512)
    ones_i = jnp.ones((16,), jnp.int32)
    zeros_i = jnp.zeros((16,), jnp.int32)
    ubase = [lane_base + u * _HIST for u in range(_UNROLL)]

    if l == 0:
        st = _init_state()
    else:
        _merge_prev(prev_c, mp_c, red_c)
        if l == 1:
            st = _init_state()
        else:
            pltpu.sync_copy(st_i_in.at[0], stv_i)
            st = _unpack_state(lane, stv_i)
        st = _select_level(l - 1, lane, red_c, st)
        stv_i[...] = _pack_state(lane, st)

    P = st[0]
    shift = 24 - 8 * l
    mask_hi = _TOPBIT if l == 0 else -(1 << (32 - 8 * l))
    shift_v = jnp.full((16,), shift, jnp.int32)

    def _zero(i, _):
        hist_c[pl.ds(i * 16, 16)] = zeros_i
        return 0

    lax.fori_loop(0, _UNROLL * _HIST // 16, _zero, 0)

    Pn, Pp = P[0], P[1]

    def _scan_one(buf, carry):
        def _scan(i, _):
            vo = i * (16 * _UNROLL)
            idxs, ms_ = [], []
            for u in range(_UNROLL):
                x = buf[pl.ds(vo + u * 16, 16)]
                mn = ((x ^ Pn) & mask_hi) == 0
                mp = ((x ^ Pp) & mask_hi) == 0
                b = lax.shift_right_logical(x, shift_v) & jnp.int32(0xFF)
                idxs.append(ubase[u] + b
                            + jnp.where(mp, jnp.int32(256), jnp.int32(0)))
                ms_.append(mn | mp)
            for u in range(_UNROLL):
                plsc.addupdate_scatter(hist_c, [idxs[u]], ones_i, mask=ms_[u])
            return 0

        lax.fori_loop(0, _VECS // _UNROLL, _scan, 0)
        return carry

    _stream_chunks(keys_hbm, wid, buf0, buf1, sem0, sem1, _scan_one, 0)

    def _fold(i, _):
        o = i * 16
        hist_c[pl.ds(o, 16)] = (hist_c[pl.ds(o, 16)] + hist_c[pl.ds(o + _HIST, 16)]
                                + hist_c[pl.ds(o + 2 * _HIST, 16)]
                                + hist_c[pl.ds(o + 3 * _HIST, 16)])
        return 0

    lax.fori_loop(0, _HIST // 16, _fold, 0)

    def _lred(j, _):
        def _acc(ln, ac):
            return ac + hist_c[pl.ds(ln * jnp.int32(512) + j * 16, 16)]

        red_c[pl.ds(j * 16, 16)] = lax.fori_loop(0, 16, _acc, zeros_i)
        return 0

    lax.fori_loop(0, 32, _lred, 0)

    pltpu.sync_copy(red_c, stage_c.at[sid])
    plsc.subcore_barrier()

    @pl.when(sid == 0)
    def _():
        def _gagg(j, _):
            def _acc(s, ac):
                return ac + mc[s, pl.ds(j * 16, 16)]

            red_c[pl.ds(j * 16, 16)] = lax.fori_loop(0, 16, _acc, zeros_i)
            return 0

        pltpu.sync_copy(stage_c, mc)
        lax.fori_loop(0, 32, _gagg, 0)
        pltpu.sync_copy(red_c, out_c.at[cid])
        if l > 0:
            pltpu.sync_copy(stv_i, st_i_out.at[cid])


def _sum_body(keys_hbm, prev_c, st_i_in, out_f,
              buf0, buf1, red_c, mp_c, stv_i, partv, mpf, stage_f, sem0, sem1):
    sid = lax.axis_index("s")
    cid = lax.axis_index("c")
    wid = cid * _NSUB + sid
    lane = lax.iota(jnp.int32, 16)
    zeros_f = jnp.zeros((16,), jnp.float32)

    _merge_prev(prev_c, mp_c, red_c)
    pltpu.sync_copy(st_i_in.at[0], stv_i)
    st = _unpack_state(lane, stv_i)
    P, c_above, kk, alive = _select_level(3, lane, red_c, st)
    tn_s = P[0] ^ _TOPBIT
    tp_s = P[1] ^ _TOPBIT

    def _scan_one(buf, carry):
        def _scan(i, acc):
            accn, accp = acc
            for u in range(_UNROLL):
                x = buf[pl.ds(i * (16 * _UNROLL) + u * 16, 16)]
                xs = x ^ _TOPBIT
                gtn = (xs > tn_s) & (xs < 0)
                gtp = xs > tp_s
                loss = plsc.bitcast(x & _MASK31, jnp.float32)
                accn = accn + jnp.where(gtn, loss, jnp.float32(0.0))
                accp = accp + jnp.where(gtp, loss, jnp.float32(0.0))
            return accn, accp

        return lax.fori_loop(0, _VECS // _UNROLL, _scan, carry)

    accn, accp = _stream_chunks(keys_hbm, wid, buf0, buf1, sem0, sem1,
                                _scan_one, (zeros_f, zeros_f))

    sv = jnp.where(lane == 0, jnp.sum(accn), zeros_f)
    sv = jnp.where(lane == 1, jnp.sum(accp), sv)
    partv[...] = sv
    pltpu.sync_copy(partv, stage_f.at[sid])
    plsc.subcore_barrier()

    @pl.when(sid == 0)
    def _():
        pltpu.sync_copy(stage_f, mpf)
        tot = lax.fori_loop(0, 16, lambda s, ac: ac + mpf[s, pl.ds(0, 16)],
                            zeros_f)
        for cls in range(2):
            t_bits = jnp.zeros((16,), jnp.int32) + (P[cls] & _MASK31)
            t_f = plsc.bitcast(t_bits, jnp.float32)
            tot = jnp.where(lane == 2 + cls, jnp.max(t_f), tot)
            tot = jnp.where(lane == 4 + cls, kk[cls].astype(jnp.float32), tot)
            tot = jnp.where(lane == 6 + cls, c_above[cls].astype(jnp.float32), tot)
            tot = jnp.where(lane == 8 + cls, alive[cls].astype(jnp.float32), tot)
        partv[...] = tot
        pltpu.sync_copy(partv, out_f.at[cid])


def _final_body(part_f, out_hbm, v0, v1, outv):
    sid = lax.axis_index("s")
    cid = lax.axis_index("c")
    lane = lax.iota(jnp.int32, 16)

    @pl.when((sid == 0) & (cid == 0))
    def _():
        pltpu.sync_copy(part_f.at[0], v0)
        pltpu.sync_copy(part_f.at[1], v1)
        a = v0[...]
        b = v1[...]
        num = jnp.zeros((16,), jnp.float32)
        den = jnp.zeros((), jnp.float32)
        for cls in range(2):
            s_tot = _lane_f(lane, a, cls) + _lane_f(lane, b, cls)
            t_f = _lane_f(lane, a, 2 + cls)
            k_f = _lane_f(lane, a, 4 + cls)
            c_f = _lane_f(lane, a, 6 + cls)
            af = _lane_f(lane, a, 8 + cls)
            num = num + af * (s_tot + (k_f - c_f) * t_f)
            den = den + af * k_f
        outv[...] = num / den
        pltpu.sync_copy(outv, out_hbm)


def _sc_select(keys):
    mesh = plsc.VectorSubcoreMesh(core_axis_name="c", subcore_axis_name="s",
                                  num_cores=_NCORE)
    params = pltpu.CompilerParams(needs_layout_passes=False)
    scan_scratch = [
        pltpu.VMEM((_CHUNK,), jnp.int32),
        pltpu.VMEM((_CHUNK,), jnp.int32),
        pltpu.VMEM((_UNROLL * _HIST,), jnp.int32),
        pltpu.VMEM((512,), jnp.int32),
        pltpu.VMEM((2, 512), jnp.int32),
        pltpu.VMEM((16, 512), jnp.int32),
        pltpu.VMEM((16,), jnp.int32),
        pltpu.VMEM_SHARED((16, 512), jnp.int32),
        pltpu.SemaphoreType.DMA,
        pltpu.SemaphoreType.DMA,
    ]

    zc = jnp.zeros((_NCORE, 512), jnp.int32)
    zi = jnp.zeros((_NCORE, 16), jnp.int32)

    hc = None
    sti = zi
    for l in range(4):
        f = pl.kernel(
            functools.partial(_scan_body, l),
            out_type=(jax.ShapeDtypeStruct((_NCORE, 512), jnp.int32),
                      jax.ShapeDtypeStruct((_NCORE, 16), jnp.int32)),
            mesh=mesh,
            compiler_params=params,
            scratch_types=scan_scratch,
        )
        hc, sti_n = f(keys, zc if hc is None else hc, sti)
        if l > 0:
            sti = sti_n

    f = pl.kernel(
        _sum_body,
        out_type=jax.ShapeDtypeStruct((_NCORE, 16), jnp.float32),
        mesh=mesh,
        compiler_params=params,
        scratch_types=[
            pltpu.VMEM((_CHUNK,), jnp.int32),
            pltpu.VMEM((_CHUNK,), jnp.int32),
            pltpu.VMEM((512,), jnp.int32),
            pltpu.VMEM((2, 512), jnp.int32),
            pltpu.VMEM((16,), jnp.int32),
            pltpu.VMEM((16,), jnp.float32),
            pltpu.VMEM((16, 16), jnp.float32),
            pltpu.VMEM_SHARED((16, 16), jnp.float32),
            pltpu.SemaphoreType.DMA,
            pltpu.SemaphoreType.DMA,
        ],
    )
    parts = f(keys, hc, sti)

    f = pl.kernel(
        _final_body,
        out_type=jax.ShapeDtypeStruct((16,), jnp.float32),
        mesh=mesh,
        compiler_params=params,
        scratch_types=[
            pltpu.VMEM((16,), jnp.float32),
            pltpu.VMEM((16,), jnp.float32),
            pltpu.VMEM((16,), jnp.float32),
        ],
    )
    return f(parts)


def kernel(inputs, targets):
    keys = _compute_keys(inputs, targets)
    out = _sc_select(keys)
    return out[0]
